# Initial kernel scaffold; baseline (speedup 1.0000x reference)
import jax
import jax.numpy as jnp
from jax.experimental import pallas as pl


def kernel(input, target):
    raise NotImplementedError("write your pallas kernel here")


# speedup vs baseline: 14.2055x; 14.2055x over previous
import functools

import jax
import jax.numpy as jnp
from jax import lax
from jax.experimental import pallas as pl
from jax.experimental.pallas import tpu as pltpu
from jax.experimental.pallas import tpu_sc as plsc

_GAMMA = 2.0
_ALPHA = 0.25

_N = 100000
_C = 80
_CP = 128
_RB = 1000
_NB = _N // _RB
_TOTAL = _N * _CP
_REAL = _N * _C
_PAD = _TOTAL - _REAL
_K = max(int(0.3 * _REAL), 1)

_BINS = 16384
_SHIFT = 17

_NC, _NS = 2, 16
_NW = _NC * _NS
_PER_W = _TOTAL // _NW
_CHUNK = 4000
_NCH = _PER_W // _CHUNK


def _loss_body(x_ref, t_ref, loss_ref, sum_ref):
    x = x_ref[...]
    t = t_ref[...]
    cls = lax.broadcasted_iota(jnp.int32, (_RB, _CP), 1)
    tt = jnp.where(cls == t, 1.0, 0.0)
    ps = jax.nn.sigmoid(x)
    pt = (1.0 - ps) * tt + ps * (1.0 - tt)
    fw = (_ALPHA * tt + (1.0 - _ALPHA) * (1.0 - tt)) * pt * pt
    bce = jnp.maximum(x, 0.0) - x * tt + jnp.log1p(jnp.exp(-jnp.abs(x)))
    loss = jnp.where(cls < _C, bce * fw, 0.0)
    loss_ref[...] = loss

    @pl.when(pl.program_id(0) == 0)
    def _():
        sum_ref[...] = jnp.zeros((1, 1), jnp.float32)

    sum_ref[...] += jnp.sum(loss, keepdims=True)


def _loss_call(x, t):
    return pl.pallas_call(
        _loss_body,
        grid=(_NB,),
        in_specs=[
            pl.BlockSpec((_RB, _CP), lambda i: (i, 0)),
            pl.BlockSpec((_RB, 1), lambda i: (i, 0)),
        ],
        out_specs=[
            pl.BlockSpec((_RB, _CP), lambda i: (i, 0)),
            pl.BlockSpec((1, 1), lambda i: (0, 0)),
        ],
        out_shape=[
            jax.ShapeDtypeStruct((_N, _CP), jnp.float32),
            jax.ShapeDtypeStruct((1, 1), jnp.float32),
        ],
    )(x, t)


def _hist_body(loss_hbm, cnt_hbm, sum_hbm, buf, hcnt, hsum):
    wid = lax.axis_index("s") * _NC + lax.axis_index("c")
    base = wid * _PER_W
    zeros = jnp.zeros((16,), jnp.float32)
    ones = jnp.ones((16,), jnp.float32)

    def zero_body(i, carry):
        hcnt[pl.ds(i * 16, 16)] = zeros
        hsum[pl.ds(i * 16, 16)] = zeros
        return carry

    lax.fori_loop(0, _BINS // 16, zero_body, 0)

    def chunk_body(ci, carry):
        pltpu.sync_copy(loss_hbm.at[pl.ds(base + ci * _CHUNK, _CHUNK)], buf)

        def vec_body(j, c2):
            v = buf[pl.ds(j * 16, 16)]
            idx = lax.shift_right_logical(plsc.bitcast(v, jnp.int32), _SHIFT)
            plsc.addupdate_scatter(hcnt, [idx], ones)
            plsc.addupdate_scatter(hsum, [idx], v)
            return c2

        lax.fori_loop(0, _CHUNK // 16, vec_body, 0)
        return carry

    lax.fori_loop(0, _NCH, chunk_body, 0)
    pltpu.sync_copy(hcnt, cnt_hbm.at[wid])
    pltpu.sync_copy(hsum, sum_hbm.at[wid])


@functools.cache
def _make_hist_call():
    return functools.partial(
        pl.kernel,
        mesh=plsc.VectorSubcoreMesh(core_axis_name="c", subcore_axis_name="s"),
        out_type=[
            jax.ShapeDtypeStruct((_NW, _BINS), jnp.float32),
            jax.ShapeDtypeStruct((_NW, _BINS), jnp.float32),
        ],
        scratch_types=[
            pltpu.VMEM((_CHUNK,), jnp.float32),
            pltpu.VMEM((_BINS,), jnp.float32),
            pltpu.VMEM((_BINS,), jnp.float32),
        ],
        compiler_params=pltpu.CompilerParams(needs_layout_passes=False),
    )(_hist_body)


_SQ = 128


def _select_body(cnt_ref, sum_ref, tot_ref, out_ref):
    h = jnp.zeros((_SQ, _SQ), jnp.float32)
    s = jnp.zeros((_SQ, _SQ), jnp.float32)
    for w in range(_NW):
        h = h + cnt_ref[w]
        s = s + sum_ref[w]
    rows = lax.broadcasted_iota(jnp.int32, (_SQ, _SQ), 0)
    cols = lax.broadcasted_iota(jnp.int32, (_SQ, _SQ), 1)
    binid = rows * _SQ + cols
    h = h - jnp.where(binid == 0, float(_PAD), 0.0)

    hi = jax.lax.Precision.HIGHEST
    inc = jnp.where(rows <= cols, 1.0, 0.0)
    strict = jnp.where(cols < rows, 1.0, 0.0)

    def csum(m):
        prefix = jax.lax.dot(m, inc, precision=hi)
        row_tot = jnp.sum(m, axis=1, keepdims=True)
        prev = jax.lax.dot(strict, row_tot, precision=hi)
        return prefix + prev

    csum_h = csum(h)
    csum_s = csum(s)
    cnt_ge = float(_REAL) - csum_h + h
    bstar = jnp.sum((cnt_ge >= float(_K)).astype(jnp.int32)) - 1
    sel = binid == bstar
    hb = jnp.sum(jnp.where(sel, h, 0.0))
    sb = jnp.sum(jnp.where(sel, s, 0.0))
    csum_hb = jnp.sum(jnp.where(sel, csum_h, 0.0))
    csum_sb = jnp.sum(jnp.where(sel, csum_s, 0.0))
    cnt_gt = float(_REAL) - csum_hb
    sum_gt = jnp.sum(s) - csum_sb
    r = float(_K) - cnt_gt
    vb = sb / jnp.maximum(hb, 1.0)
    topk_sum = sum_gt + r * vb
    out_ref[...] = tot_ref[...] / float(_REAL) + jnp.full((1, 1), topk_sum / float(_K))


def _select_call(cnt, sm, tot):
    return pl.pallas_call(
        _select_body,
        out_shape=jax.ShapeDtypeStruct((1, 1), jnp.float32),
    )(cnt.reshape(_NW, _SQ, _SQ), sm.reshape(_NW, _SQ, _SQ), tot)


def kernel(input, target):
    x = jnp.pad(input, ((0, 0), (0, _CP - _C)))
    t2 = target.reshape(_N, 1)
    loss, tot = _loss_call(x, t2)
    cnt, sm = _make_hist_call()(loss.reshape(_TOTAL))
    res = _select_call(cnt, sm, tot)
    return res[0, 0]


# speedup vs baseline: 34.4572x; 2.4256x over previous
import functools

import jax
import jax.numpy as jnp
from jax import lax
from jax.experimental import pallas as pl
from jax.experimental.pallas import tpu as pltpu
from jax.experimental.pallas import tpu_sc as plsc

_GAMMA = 2.0
_ALPHA = 0.25

_N = 100000
_C = 80
_CP = 128
_RB = 1000
_NB = _N // _RB
_TOTAL = _N * _CP
_REAL = _N * _C
_PAD = _TOTAL - _REAL
_K = max(int(0.3 * _REAL), 1)

_BINS = 16384
_SHIFT = 17

_NC, _NS = 2, 16
_NW = _NC * _NS
_NRC = 80
_NCHT = _N // _NRC
_PER_CH = -(-_NCHT // _NW)


def _loss_body(x_ref, t_ref, loss_ref, sum_ref):
    x = x_ref[...]
    t = t_ref[...]
    cls = lax.broadcasted_iota(jnp.int32, (_RB, _CP), 1)
    tt = jnp.where(cls == t, 1.0, 0.0)
    ps = jax.nn.sigmoid(x)
    pt = (1.0 - ps) * tt + ps * (1.0 - tt)
    fw = (_ALPHA * tt + (1.0 - _ALPHA) * (1.0 - tt)) * pt * pt
    bce = jnp.maximum(x, 0.0) - x * tt + jnp.log1p(jnp.exp(-jnp.abs(x)))
    loss = jnp.where(cls < _C, bce * fw, 0.0)
    loss_ref[...] = loss

    @pl.when(pl.program_id(0) == 0)
    def _():
        sum_ref[...] = jnp.zeros((1, 1), jnp.float32)

    sum_ref[...] += jnp.sum(loss, keepdims=True)


def _loss_call(x, t):
    return pl.pallas_call(
        _loss_body,
        grid=(_NB,),
        in_specs=[
            pl.BlockSpec((_RB, _CP), lambda i: (i, 0)),
            pl.BlockSpec((_RB, 1), lambda i: (i, 0)),
        ],
        out_specs=[
            pl.BlockSpec((_RB, _CP), lambda i: (i, 0)),
            pl.BlockSpec((1, 1), lambda i: (0, 0)),
        ],
        out_shape=[
            jax.ShapeDtypeStruct((_N, _CP), jnp.float32),
            jax.ShapeDtypeStruct((1, 1), jnp.float32),
        ],
    )(x, t)


def _hist_body(loss_hbm, cnt_hbm, sum_hbm, buf, hcnt, hsum, sem0, sem1):
    wid = lax.axis_index("s") * _NC + lax.axis_index("c")
    lo = jnp.minimum(wid * _PER_CH, _NCHT)
    n = jnp.minimum(lo + _PER_CH, _NCHT) - lo
    zeros = jnp.zeros((16,), jnp.float32)
    ones = jnp.ones((16,), jnp.float32)

    @plsc.parallel_loop(0, _BINS // 16, unroll=8)
    def _(i):
        hcnt[pl.ds(i * 16, 16)] = zeros
        hsum[pl.ds(i * 16, 16)] = zeros

    sems = [sem0, sem1]

    def start(ci, slot):
        pltpu.async_copy(loss_hbm.at[pl.ds((lo + ci) * _NRC, _NRC)],
                         buf.at[slot], sems[slot])

    def wait(slot):
        pltpu.make_async_copy(loss_hbm.at[pl.ds(0, _NRC)], buf.at[slot],
                              sems[slot]).wait()

    def process(slot):
        @plsc.parallel_loop(0, _NRC, unroll=2)
        def _(r):
            for sub in range(_C // 16):
                v = buf[slot, r, pl.ds(sub * 16, 16)]
                idx = lax.shift_right_logical(plsc.bitcast(v, jnp.int32),
                                              _SHIFT)
                plsc.addupdate_scatter(hcnt, [idx], ones)
                plsc.addupdate_scatter(hsum, [idx], v)

    @pl.when(n > 0)
    def _():
        start(0, 0)

    @pl.when(n > 1)
    def _():
        start(1, 1)

    def outer(g, c):
        for b in range(2):
            ci = g * 2 + b

            @pl.when(ci < n)
            def _():
                wait(b)
                process(b)

                @pl.when(ci + 2 < n)
                def _():
                    start(ci + 2, b)
        return c

    lax.fori_loop(0, _PER_CH // 2, outer, 0)
    pltpu.sync_copy(hcnt, cnt_hbm.at[wid])
    pltpu.sync_copy(hsum, sum_hbm.at[wid])


@functools.cache
def _make_hist_call():
    return functools.partial(
        pl.kernel,
        mesh=plsc.VectorSubcoreMesh(core_axis_name="c", subcore_axis_name="s"),
        out_type=[
            jax.ShapeDtypeStruct((_NW, _BINS), jnp.float32),
            jax.ShapeDtypeStruct((_NW, _BINS), jnp.float32),
        ],
        scratch_types=[
            pltpu.VMEM((2, _NRC, _CP), jnp.float32),
            pltpu.VMEM((_BINS,), jnp.float32),
            pltpu.VMEM((_BINS,), jnp.float32),
            pltpu.SemaphoreType.DMA,
            pltpu.SemaphoreType.DMA,
        ],
        compiler_params=pltpu.CompilerParams(needs_layout_passes=False),
    )(_hist_body)


_SQ = 128


def _select_body(cnt_ref, sum_ref, tot_ref, out_ref):
    h = jnp.zeros((_SQ, _SQ), jnp.float32)
    s = jnp.zeros((_SQ, _SQ), jnp.float32)
    for w in range(_NW):
        h = h + cnt_ref[w]
        s = s + sum_ref[w]
    rows = lax.broadcasted_iota(jnp.int32, (_SQ, _SQ), 0)
    cols = lax.broadcasted_iota(jnp.int32, (_SQ, _SQ), 1)
    binid = rows * _SQ + cols

    hi = jax.lax.Precision.HIGHEST
    inc = jnp.where(rows <= cols, 1.0, 0.0)
    strict = jnp.where(cols < rows, 1.0, 0.0)

    def csum(m):
        prefix = jax.lax.dot(m, inc, precision=hi)
        row_tot = jnp.sum(m, axis=1, keepdims=True)
        prev = jax.lax.dot(strict, row_tot, precision=hi)
        return prefix + prev

    csum_h = csum(h)
    csum_s = csum(s)
    cnt_ge = float(_REAL) - csum_h + h
    bstar = jnp.sum((cnt_ge >= float(_K)).astype(jnp.int32)) - 1
    sel = binid == bstar
    hb = jnp.sum(jnp.where(sel, h, 0.0))
    sb = jnp.sum(jnp.where(sel, s, 0.0))
    csum_hb = jnp.sum(jnp.where(sel, csum_h, 0.0))
    csum_sb = jnp.sum(jnp.where(sel, csum_s, 0.0))
    cnt_gt = float(_REAL) - csum_hb
    sum_gt = jnp.sum(s) - csum_sb
    r = float(_K) - cnt_gt
    vb = sb / jnp.maximum(hb, 1.0)
    topk_sum = sum_gt + r * vb
    out_ref[...] = tot_ref[...] / float(_REAL) + jnp.full((1, 1), topk_sum / float(_K))


def _select_call(cnt, sm, tot):
    return pl.pallas_call(
        _select_body,
        out_shape=jax.ShapeDtypeStruct((1, 1), jnp.float32),
    )(cnt.reshape(_NW, _SQ, _SQ), sm.reshape(_NW, _SQ, _SQ), tot)


def kernel(input, target):
    x = jnp.pad(input, ((0, 0), (0, _CP - _C)))
    t2 = target.reshape(_N, 1)
    loss, tot = _loss_call(x, t2)
    cnt, sm = _make_hist_call()(loss)
    res = _select_call(cnt, sm, tot)
    return res[0, 0]


# speedup vs baseline: 45.1192x; 1.3094x over previous
import functools

import jax
import jax.numpy as jnp
from jax import lax
from jax.experimental import pallas as pl
from jax.experimental.pallas import tpu as pltpu
from jax.experimental.pallas import tpu_sc as plsc

_GAMMA = 2.0
_ALPHA = 0.25

_N = 100000
_C = 80
_CP = 128
_RB = 1000
_NB = _N // _RB
_TOTAL = _N * _CP
_REAL = _N * _C
_PAD = _TOTAL - _REAL
_K = max(int(0.3 * _REAL), 1)

_BINS = 16384
_SHIFT = 17

_NC, _NS = 2, 16
_NW = _NC * _NS
_NRC = 80
_NCHT = _N // _NRC
_PER_CH = -(-_NCHT // _NW)


def _loss_body(x_ref, t_ref, loss_ref, sum_ref):
    x = x_ref[...]
    t = t_ref[...]
    cls = lax.broadcasted_iota(jnp.int32, (_RB, _C), 1)
    tt = jnp.where(cls == t, 1.0, 0.0)
    ps = jax.nn.sigmoid(x)
    pt = (1.0 - ps) * tt + ps * (1.0 - tt)
    fw = (_ALPHA * tt + (1.0 - _ALPHA) * (1.0 - tt)) * pt * pt
    bce = jnp.maximum(x, 0.0) - x * tt + jnp.log1p(jnp.exp(-jnp.abs(x)))
    loss = jnp.concatenate(
        [bce * fw, jnp.zeros((_RB, _CP - _C), jnp.float32)], axis=1)
    loss_ref[...] = loss

    @pl.when(pl.program_id(0) == 0)
    def _():
        sum_ref[...] = jnp.zeros((1, 1), jnp.float32)

    sum_ref[...] += jnp.sum(loss, keepdims=True)


def _loss_call(x, t):
    return pl.pallas_call(
        _loss_body,
        grid=(_NB,),
        in_specs=[
            pl.BlockSpec((_RB, _C), lambda i: (i, 0)),
            pl.BlockSpec((_RB, 1), lambda i: (i, 0)),
        ],
        out_specs=[
            pl.BlockSpec((_RB, _CP), lambda i: (i, 0)),
            pl.BlockSpec((1, 1), lambda i: (0, 0)),
        ],
        out_shape=[
            jax.ShapeDtypeStruct((_N, _CP), jnp.float32),
            jax.ShapeDtypeStruct((1, 1), jnp.float32),
        ],
    )(x, t)


def _hist_body(loss_hbm, cnt_hbm, sum_hbm, buf, hcnt, hsum, sem0, sem1):
    wid = lax.axis_index("s") * _NC + lax.axis_index("c")
    lo = jnp.minimum(wid * _PER_CH, _NCHT)
    n = jnp.minimum(lo + _PER_CH, _NCHT) - lo
    zeros = jnp.zeros((16,), jnp.float32)
    ones = jnp.ones((16,), jnp.float32)

    @plsc.parallel_loop(0, _BINS // 16, unroll=8)
    def _(i):
        hcnt[pl.ds(i * 16, 16)] = zeros
        hsum[pl.ds(i * 16, 16)] = zeros

    sems = [sem0, sem1]

    def start(ci, slot):
        pltpu.async_copy(loss_hbm.at[pl.ds((lo + ci) * _NRC, _NRC)],
                         buf.at[slot], sems[slot])

    def wait(slot):
        pltpu.make_async_copy(loss_hbm.at[pl.ds(0, _NRC)], buf.at[slot],
                              sems[slot]).wait()

    def process(slot):
        @plsc.parallel_loop(0, _NRC, unroll=2)
        def _(r):
            for sub in range(_C // 16):
                v = buf[slot, r, pl.ds(sub * 16, 16)]
                idx = lax.shift_right_logical(plsc.bitcast(v, jnp.int32),
                                              _SHIFT)
                plsc.addupdate_scatter(hcnt, [idx], ones)
                plsc.addupdate_scatter(hsum, [idx], v)

    @pl.when(n > 0)
    def _():
        start(0, 0)

    @pl.when(n > 1)
    def _():
        start(1, 1)

    def outer(g, c):
        for b in range(2):
            ci = g * 2 + b

            @pl.when(ci < n)
            def _():
                wait(b)
                process(b)

                @pl.when(ci + 2 < n)
                def _():
                    start(ci + 2, b)
        return c

    lax.fori_loop(0, _PER_CH // 2, outer, 0)
    pltpu.sync_copy(hcnt, cnt_hbm.at[wid])
    pltpu.sync_copy(hsum, sum_hbm.at[wid])


@functools.cache
def _make_hist_call():
    return functools.partial(
        pl.kernel,
        mesh=plsc.VectorSubcoreMesh(core_axis_name="c", subcore_axis_name="s"),
        out_type=[
            jax.ShapeDtypeStruct((_NW, _BINS), jnp.float32),
            jax.ShapeDtypeStruct((_NW, _BINS), jnp.float32),
        ],
        scratch_types=[
            pltpu.VMEM((2, _NRC, _CP), jnp.float32),
            pltpu.VMEM((_BINS,), jnp.float32),
            pltpu.VMEM((_BINS,), jnp.float32),
            pltpu.SemaphoreType.DMA,
            pltpu.SemaphoreType.DMA,
        ],
        compiler_params=pltpu.CompilerParams(needs_layout_passes=False),
    )(_hist_body)


_SQ = 128


def _select_body(cnt_ref, sum_ref, tot_ref, out_ref):
    h = jnp.zeros((_SQ, _SQ), jnp.float32)
    s = jnp.zeros((_SQ, _SQ), jnp.float32)
    for w in range(_NW):
        h = h + cnt_ref[w]
        s = s + sum_ref[w]
    rows = lax.broadcasted_iota(jnp.int32, (_SQ, _SQ), 0)
    cols = lax.broadcasted_iota(jnp.int32, (_SQ, _SQ), 1)
    binid = rows * _SQ + cols

    hi = jax.lax.Precision.HIGHEST
    inc = jnp.where(rows <= cols, 1.0, 0.0)
    strict = jnp.where(cols < rows, 1.0, 0.0)

    def csum(m):
        prefix = jax.lax.dot(m, inc, precision=hi)
        row_tot = jnp.sum(m, axis=1, keepdims=True)
        prev = jax.lax.dot(strict, row_tot, precision=hi)
        return prefix + prev

    csum_h = csum(h)
    csum_s = csum(s)
    cnt_ge = float(_REAL) - csum_h + h
    bstar = jnp.sum((cnt_ge >= float(_K)).astype(jnp.int32)) - 1
    sel = binid == bstar
    hb = jnp.sum(jnp.where(sel, h, 0.0))
    sb = jnp.sum(jnp.where(sel, s, 0.0))
    csum_hb = jnp.sum(jnp.where(sel, csum_h, 0.0))
    csum_sb = jnp.sum(jnp.where(sel, csum_s, 0.0))
    cnt_gt = float(_REAL) - csum_hb
    sum_gt = jnp.sum(s) - csum_sb
    r = float(_K) - cnt_gt
    vb = sb / jnp.maximum(hb, 1.0)
    topk_sum = sum_gt + r * vb
    out_ref[...] = tot_ref[...] / float(_REAL) + jnp.full((1, 1), topk_sum / float(_K))


def _select_call(cnt, sm, tot):
    return pl.pallas_call(
        _select_body,
        out_shape=jax.ShapeDtypeStruct((1, 1), jnp.float32),
    )(cnt.reshape(_NW, _SQ, _SQ), sm.reshape(_NW, _SQ, _SQ), tot)


def kernel(input, target):
    t2 = target.reshape(_N, 1)
    loss, tot = _loss_call(input, t2)
    cnt, sm = _make_hist_call()(loss)
    res = _select_call(cnt, sm, tot)
    return res[0, 0]


# speedup vs baseline: 45.2858x; 1.0037x over previous
import functools

import jax
import jax.numpy as jnp
from jax import lax
from jax.experimental import pallas as pl
from jax.experimental.pallas import tpu as pltpu
from jax.experimental.pallas import tpu_sc as plsc

_GAMMA = 2.0
_ALPHA = 0.25

_N = 100000
_C = 80
_CP = 128
_RB = 1000
_NB = _N // _RB
_TOTAL = _N * _CP
_REAL = _N * _C
_PAD = _TOTAL - _REAL
_K = max(int(0.3 * _REAL), 1)

_BINS = 16384
_SHIFT = 17

_NC, _NS = 2, 16
_NW = _NC * _NS
_NRC = 80
_NCHT = _N // _NRC
_PER_CH = -(-_NCHT // _NW)


def _loss_body(x_ref, t_ref, loss_ref, sum_ref):
    x = x_ref[...]
    t = t_ref[...]
    cls = lax.broadcasted_iota(jnp.int32, (_RB, _C), 1)
    tt = jnp.where(cls == t, 1.0, 0.0)
    ps = jax.nn.sigmoid(x)
    pt = (1.0 - ps) * tt + ps * (1.0 - tt)
    fw = (_ALPHA * tt + (1.0 - _ALPHA) * (1.0 - tt)) * pt * pt
    bce = jnp.maximum(x, 0.0) - x * tt + jnp.log1p(jnp.exp(-jnp.abs(x)))
    loss = bce * fw
    loss_ref[...] = loss

    @pl.when(pl.program_id(0) == 0)
    def _():
        sum_ref[...] = jnp.zeros((1, 1), jnp.float32)

    sum_ref[...] += jnp.sum(loss, keepdims=True)


def _loss_call(x, t):
    return pl.pallas_call(
        _loss_body,
        grid=(_NB,),
        in_specs=[
            pl.BlockSpec((_RB, _C), lambda i: (i, 0)),
            pl.BlockSpec((_RB, 1), lambda i: (i, 0)),
        ],
        out_specs=[
            pl.BlockSpec((_RB, _C), lambda i: (i, 0)),
            pl.BlockSpec((1, 1), lambda i: (0, 0)),
        ],
        out_shape=[
            jax.ShapeDtypeStruct((_N, _C), jnp.float32),
            jax.ShapeDtypeStruct((1, 1), jnp.float32),
        ],
    )(x, t)


def _hist_body(loss_hbm, cnt_hbm, sum_hbm, buf, hcnt, hsum, sem0, sem1):
    wid = lax.axis_index("s") * _NC + lax.axis_index("c")
    lo = jnp.minimum(wid * _PER_CH, _NCHT)
    n = jnp.minimum(lo + _PER_CH, _NCHT) - lo
    zeros = jnp.zeros((16,), jnp.float32)
    ones = jnp.ones((16,), jnp.float32)

    @plsc.parallel_loop(0, _BINS // 16, unroll=8)
    def _(i):
        hcnt[pl.ds(i * 16, 16)] = zeros
        hsum[pl.ds(i * 16, 16)] = zeros

    sems = [sem0, sem1]

    def start(ci, slot):
        pltpu.async_copy(loss_hbm.at[pl.ds((lo + ci) * _NRC, _NRC)],
                         buf.at[slot], sems[slot])

    def wait(slot):
        pltpu.make_async_copy(loss_hbm.at[pl.ds(0, _NRC)], buf.at[slot],
                              sems[slot]).wait()

    def process(slot):
        @plsc.parallel_loop(0, _NRC, unroll=2)
        def _(r):
            for sub in range(_C // 16):
                v = buf[slot, r, pl.ds(sub * 16, 16)]
                idx = lax.shift_right_logical(plsc.bitcast(v, jnp.int32),
                                              _SHIFT)
                plsc.addupdate_scatter(hcnt, [idx], ones)
                plsc.addupdate_scatter(hsum, [idx], v)

    @pl.when(n > 0)
    def _():
        start(0, 0)

    @pl.when(n > 1)
    def _():
        start(1, 1)

    def outer(g, c):
        for b in range(2):
            ci = g * 2 + b

            @pl.when(ci < n)
            def _():
                wait(b)
                process(b)

                @pl.when(ci + 2 < n)
                def _():
                    start(ci + 2, b)
        return c

    lax.fori_loop(0, _PER_CH // 2, outer, 0)
    pltpu.sync_copy(hcnt, cnt_hbm.at[wid])
    pltpu.sync_copy(hsum, sum_hbm.at[wid])


@functools.cache
def _make_hist_call():
    return functools.partial(
        pl.kernel,
        mesh=plsc.VectorSubcoreMesh(core_axis_name="c", subcore_axis_name="s"),
        out_type=[
            jax.ShapeDtypeStruct((_NW, _BINS), jnp.float32),
            jax.ShapeDtypeStruct((_NW, _BINS), jnp.float32),
        ],
        scratch_types=[
            pltpu.VMEM((2, _NRC, _C), jnp.float32),
            pltpu.VMEM((_BINS,), jnp.float32),
            pltpu.VMEM((_BINS,), jnp.float32),
            pltpu.SemaphoreType.DMA,
            pltpu.SemaphoreType.DMA,
        ],
        compiler_params=pltpu.CompilerParams(needs_layout_passes=False),
    )(_hist_body)


_SQ = 128


def _select_body(cnt_ref, sum_ref, tot_ref, out_ref):
    h = jnp.zeros((_SQ, _SQ), jnp.float32)
    s = jnp.zeros((_SQ, _SQ), jnp.float32)
    for w in range(_NW):
        h = h + cnt_ref[w]
        s = s + sum_ref[w]
    rows = lax.broadcasted_iota(jnp.int32, (_SQ, _SQ), 0)
    cols = lax.broadcasted_iota(jnp.int32, (_SQ, _SQ), 1)
    binid = rows * _SQ + cols

    hi = jax.lax.Precision.HIGHEST
    inc = jnp.where(rows <= cols, 1.0, 0.0)
    strict = jnp.where(cols < rows, 1.0, 0.0)

    def csum(m):
        prefix = jax.lax.dot(m, inc, precision=hi)
        row_tot = jnp.sum(m, axis=1, keepdims=True)
        prev = jax.lax.dot(strict, row_tot, precision=hi)
        return prefix + prev

    csum_h = csum(h)
    csum_s = csum(s)
    cnt_ge = float(_REAL) - csum_h + h
    bstar = jnp.sum((cnt_ge >= float(_K)).astype(jnp.int32)) - 1
    sel = binid == bstar
    hb = jnp.sum(jnp.where(sel, h, 0.0))
    sb = jnp.sum(jnp.where(sel, s, 0.0))
    csum_hb = jnp.sum(jnp.where(sel, csum_h, 0.0))
    csum_sb = jnp.sum(jnp.where(sel, csum_s, 0.0))
    cnt_gt = float(_REAL) - csum_hb
    sum_gt = jnp.sum(s) - csum_sb
    r = float(_K) - cnt_gt
    vb = sb / jnp.maximum(hb, 1.0)
    topk_sum = sum_gt + r * vb
    out_ref[...] = tot_ref[...] / float(_REAL) + jnp.full((1, 1), topk_sum / float(_K))


def _select_call(cnt, sm, tot):
    return pl.pallas_call(
        _select_body,
        out_shape=jax.ShapeDtypeStruct((1, 1), jnp.float32),
    )(cnt.reshape(_NW, _SQ, _SQ), sm.reshape(_NW, _SQ, _SQ), tot)


def kernel(input, target):
    t2 = target.reshape(_N, 1)
    loss, tot = _loss_call(input, t2)
    cnt, sm = _make_hist_call()(loss)
    res = _select_call(cnt, sm, tot)
    return res[0, 0]


# speedup vs baseline: 59.4872x; 1.3136x over previous
import functools

import jax
import jax.numpy as jnp
from jax import lax
from jax.experimental import pallas as pl
from jax.experimental.pallas import tpu as pltpu
from jax.experimental.pallas import tpu_sc as plsc

_ALPHA = 0.25

_N = 100000
_C = 80
_NP = 100096
_REAL = _N * _C
_PADN = (_NP - _N) * _C
_K = max(int(0.3 * _REAL), 1)

_BLK = 512
_NB = -(-_NP // _BLK)

_BINS = 16384
_SHIFT = 17
_SQ = 128

_NC, _NS = 2, 16
_NW = _NC * _NS
_CCH = 128
_NCHT = _NP // _CCH
_PER_CH = -(-_NCHT // _NW)


def _loss_body(x_ref, t_ref, loss_ref, sum_ref):
    x = x_ref[...]
    t = t_ref[...]
    cls = lax.broadcasted_iota(jnp.int32, (_C, _BLK), 0)
    tt = jnp.where(cls == t, 1.0, 0.0)
    ax = jnp.abs(x)
    e = jnp.exp(-ax)
    opp = 1.0 + e
    l = jnp.log(opp)
    ps = jnp.where(x >= 0.0, 1.0, e) / opp
    pt = ps + tt - 2.0 * ps * tt
    aw = (1.0 - _ALPHA) + (2.0 * _ALPHA - 1.0) * tt
    bce = jnp.maximum(x, 0.0) - x * tt + l
    raw = aw * (pt * pt) * bce
    col = pl.program_id(0) * _BLK + lax.broadcasted_iota(jnp.int32,
                                                         (_C, _BLK), 1)
    loss = jnp.where(col < _N, raw, 0.0)
    loss_ref[...] = loss

    @pl.when(pl.program_id(0) == 0)
    def _():
        sum_ref[...] = jnp.zeros((1, 1), jnp.float32)

    sum_ref[...] += jnp.sum(loss, keepdims=True)


def _loss_call(xt, t):
    return pl.pallas_call(
        _loss_body,
        grid=(_NB,),
        in_specs=[
            pl.BlockSpec((_C, _BLK), lambda i: (0, i)),
            pl.BlockSpec((1, _BLK), lambda i: (0, i)),
        ],
        out_specs=[
            pl.BlockSpec((_C, _BLK), lambda i: (0, i)),
            pl.BlockSpec((1, 1), lambda i: (0, 0)),
        ],
        out_shape=[
            jax.ShapeDtypeStruct((_C, _NP), jnp.float32),
            jax.ShapeDtypeStruct((1, 1), jnp.float32),
        ],
    )(xt, t)


def _hist_body(loss_hbm, cnt_hbm, sum_hbm, buf, hcnt, hsum, sem0, sem1):
    wid = lax.axis_index("s") * _NC + lax.axis_index("c")
    lo = jnp.minimum(wid * _PER_CH, _NCHT)
    n = jnp.minimum(lo + _PER_CH, _NCHT) - lo
    zeros = jnp.zeros((16,), jnp.float32)
    ones = jnp.ones((16,), jnp.float32)

    @plsc.parallel_loop(0, _SQ, unroll=8)
    def _(i):
        for sub in range(_SQ // 16):
            hcnt[i, pl.ds(sub * 16, 16)] = zeros
            hsum[i, pl.ds(sub * 16, 16)] = zeros

    sems = [sem0, sem1]

    def start(ci, slot):
        pltpu.async_copy(loss_hbm.at[:, pl.ds((lo + ci) * _CCH, _CCH)],
                         buf.at[slot], sems[slot])

    def wait(slot):
        pltpu.make_async_copy(loss_hbm.at[:, pl.ds(0, _CCH)], buf.at[slot],
                              sems[slot]).wait()

    def process(slot):
        @plsc.parallel_loop(0, _C, unroll=2)
        def _(r):
            for sub in range(_CCH // 16):
                v = buf[slot, r, pl.ds(sub * 16, 16)]
                idx = lax.shift_right_logical(plsc.bitcast(v, jnp.int32),
                                              _SHIFT)
                hi = lax.shift_right_logical(idx, 7)
                lo_i = lax.bitwise_and(idx, 127)
                plsc.addupdate_scatter(hcnt, [hi, lo_i], ones)
                plsc.addupdate_scatter(hsum, [hi, lo_i], v)

    @pl.when(n > 0)
    def _():
        start(0, 0)

    @pl.when(n > 1)
    def _():
        start(1, 1)

    def outer(g, c):
        for b in range(2):
            ci = g * 2 + b

            @pl.when(ci < n)
            def _():
                wait(b)
                process(b)

                @pl.when(ci + 2 < n)
                def _():
                    start(ci + 2, b)
        return c

    lax.fori_loop(0, (_PER_CH + 1) // 2, outer, 0)
    pltpu.sync_copy(hcnt, cnt_hbm.at[wid])
    pltpu.sync_copy(hsum, sum_hbm.at[wid])


@functools.cache
def _make_hist_call():
    return functools.partial(
        pl.kernel,
        mesh=plsc.VectorSubcoreMesh(core_axis_name="c", subcore_axis_name="s"),
        out_type=[
            jax.ShapeDtypeStruct((_NW, _SQ, _SQ), jnp.float32),
            jax.ShapeDtypeStruct((_NW, _SQ, _SQ), jnp.float32),
        ],
        scratch_types=[
            pltpu.VMEM((2, _C, _CCH), jnp.float32),
            pltpu.VMEM((_SQ, _SQ), jnp.float32),
            pltpu.VMEM((_SQ, _SQ), jnp.float32),
            pltpu.SemaphoreType.DMA,
            pltpu.SemaphoreType.DMA,
        ],
        compiler_params=pltpu.CompilerParams(needs_layout_passes=False),
    )(_hist_body)


def _select_body(cnt_ref, sum_ref, tot_ref, out_ref):
    h = jnp.zeros((_SQ, _SQ), jnp.float32)
    s = jnp.zeros((_SQ, _SQ), jnp.float32)
    for w in range(_NW):
        h = h + cnt_ref[w]
        s = s + sum_ref[w]
    rows = lax.broadcasted_iota(jnp.int32, (_SQ, _SQ), 0)
    cols = lax.broadcasted_iota(jnp.int32, (_SQ, _SQ), 1)
    binid = rows * _SQ + cols
    h = h - jnp.where(binid == 0, float(_PADN), 0.0)

    hi = jax.lax.Precision.HIGHEST
    inc = jnp.where(rows <= cols, 1.0, 0.0)
    strict = jnp.where(cols < rows, 1.0, 0.0)

    def csum(m):
        prefix = jax.lax.dot(m, inc, precision=hi)
        row_tot = jnp.sum(m, axis=1, keepdims=True)
        prev = jax.lax.dot(strict, row_tot, precision=hi)
        return prefix + prev

    csum_h = csum(h)
    csum_s = csum(s)
    cnt_ge = float(_REAL) - csum_h + h
    bstar = jnp.sum((cnt_ge >= float(_K)).astype(jnp.int32)) - 1
    sel = binid == bstar
    hb = jnp.sum(jnp.where(sel, h, 0.0))
    sb = jnp.sum(jnp.where(sel, s, 0.0))
    csum_hb = jnp.sum(jnp.where(sel, csum_h, 0.0))
    csum_sb = jnp.sum(jnp.where(sel, csum_s, 0.0))
    cnt_gt = float(_REAL) - csum_hb
    sum_gt = jnp.sum(s) - csum_sb
    r = float(_K) - cnt_gt
    vb = sb / jnp.maximum(hb, 1.0)
    topk_sum = sum_gt + r * vb
    out_ref[...] = tot_ref[...] / float(_REAL) + jnp.full((1, 1), topk_sum / float(_K))


def _select_call(cnt, sm, tot):
    return pl.pallas_call(
        _select_body,
        out_shape=jax.ShapeDtypeStruct((1, 1), jnp.float32),
    )(cnt, sm, tot)


def kernel(input, target):
    xt = input.T
    t2 = target.reshape(1, _N)
    loss, tot = _loss_call(xt, t2)
    cnt, sm = _make_hist_call()(loss)
    res = _select_call(cnt, sm, tot)
    return res[0, 0]


# speedup vs baseline: 103.9037x; 1.7467x over previous
import functools

import jax
import jax.numpy as jnp
from jax import lax
from jax.experimental import pallas as pl
from jax.experimental.pallas import tpu as pltpu
from jax.experimental.pallas import tpu_sc as plsc

_ALPHA = 0.25

_N = 100000
_C = 80
_NP = 100096
_REAL = _N * _C
_PADN = (_NP - _N) * _C
_K = max(int(0.3 * _REAL), 1)

_BLK = 4352
_NB = -(-_NP // _BLK)

_BINS = 16384
_SHIFT = 17
_SQ = 128

_NC, _NS = 2, 16
_NW = _NC * _NS
_CCH = 128
_NCHT = _NP // _CCH
_PER_CH = -(-_NCHT // _NW)


def _loss_body(x_ref, t_ref, loss_ref, sum_ref):
    x = x_ref[...]
    t = t_ref[...]
    cls = lax.broadcasted_iota(jnp.int32, (_C, _BLK), 0)
    tt = jnp.where(cls == t, 1.0, 0.0)
    ax = jnp.abs(x)
    e = jnp.exp(-ax)
    opp = 1.0 + e
    l = jnp.log(opp)
    ps = jnp.where(x >= 0.0, 1.0, e) / opp
    pt = ps + tt - 2.0 * ps * tt
    aw = (1.0 - _ALPHA) + (2.0 * _ALPHA - 1.0) * tt
    bce = jnp.maximum(x, 0.0) - x * tt + l
    raw = aw * (pt * pt) * bce
    col = pl.program_id(0) * _BLK + lax.broadcasted_iota(jnp.int32,
                                                         (_C, _BLK), 1)
    loss = jnp.where(col < _N, raw, 0.0)
    loss_ref[...] = loss

    @pl.when(pl.program_id(0) == 0)
    def _():
        sum_ref[...] = jnp.zeros((1, 1), jnp.float32)

    sum_ref[...] += jnp.sum(loss, keepdims=True)


def _loss_call(xt, t):
    return pl.pallas_call(
        _loss_body,
        grid=(_NB,),
        in_specs=[
            pl.BlockSpec((_C, _BLK), lambda i: (0, i)),
            pl.BlockSpec((1, _BLK), lambda i: (0, i)),
        ],
        out_specs=[
            pl.BlockSpec((_C, _BLK), lambda i: (0, i)),
            pl.BlockSpec((1, 1), lambda i: (0, 0)),
        ],
        out_shape=[
            jax.ShapeDtypeStruct((_C, _NP), jnp.float32),
            jax.ShapeDtypeStruct((1, 1), jnp.float32),
        ],
    )(xt, t)


def _hist_body(loss_hbm, cnt_hbm, sum_hbm, buf, hcnt, hsum, sem0, sem1):
    wid = lax.axis_index("s") * _NC + lax.axis_index("c")
    lo = jnp.minimum(wid * _PER_CH, _NCHT)
    n = jnp.minimum(lo + _PER_CH, _NCHT) - lo
    zeros = jnp.zeros((16,), jnp.float32)
    ones = jnp.ones((16,), jnp.float32)

    @plsc.parallel_loop(0, _SQ, unroll=8)
    def _(i):
        for sub in range(_SQ // 16):
            hcnt[i, pl.ds(sub * 16, 16)] = zeros
            hsum[i, pl.ds(sub * 16, 16)] = zeros

    sems = [sem0, sem1]

    def start(ci, slot):
        pltpu.async_copy(loss_hbm.at[:, pl.ds((lo + ci) * _CCH, _CCH)],
                         buf.at[slot], sems[slot])

    def wait(slot):
        pltpu.make_async_copy(loss_hbm.at[:, pl.ds(0, _CCH)], buf.at[slot],
                              sems[slot]).wait()

    def process(slot):
        @plsc.parallel_loop(0, _C, unroll=4)
        def _(r):
            for sub in range(_CCH // 16):
                v = buf[slot, r, pl.ds(sub * 16, 16)]
                idx = lax.shift_right_logical(plsc.bitcast(v, jnp.int32),
                                              _SHIFT)
                hi = lax.shift_right_logical(idx, 7)
                lo_i = lax.bitwise_and(idx, 127)
                plsc.addupdate_scatter(hcnt, [hi, lo_i], ones)
                plsc.addupdate_scatter(hsum, [hi, lo_i], v)

    @pl.when(n > 0)
    def _():
        start(0, 0)

    @pl.when(n > 1)
    def _():
        start(1, 1)

    def outer(g, c):
        for b in range(2):
            ci = g * 2 + b

            @pl.when(ci < n)
            def _():
                wait(b)
                process(b)

                @pl.when(ci + 2 < n)
                def _():
                    start(ci + 2, b)
        return c

    lax.fori_loop(0, (_PER_CH + 1) // 2, outer, 0)
    pltpu.sync_copy(hcnt, cnt_hbm.at[wid])
    pltpu.sync_copy(hsum, sum_hbm.at[wid])


@functools.cache
def _make_hist_call():
    return functools.partial(
        pl.kernel,
        mesh=plsc.VectorSubcoreMesh(core_axis_name="c", subcore_axis_name="s"),
        out_type=[
            jax.ShapeDtypeStruct((_NW, _SQ, _SQ), jnp.float32),
            jax.ShapeDtypeStruct((_NW, _SQ, _SQ), jnp.float32),
        ],
        scratch_types=[
            pltpu.VMEM((2, _C, _CCH), jnp.float32),
            pltpu.VMEM((_SQ, _SQ), jnp.float32),
            pltpu.VMEM((_SQ, _SQ), jnp.float32),
            pltpu.SemaphoreType.DMA,
            pltpu.SemaphoreType.DMA,
        ],
        compiler_params=pltpu.CompilerParams(needs_layout_passes=False),
    )(_hist_body)


def _select_body(cnt_ref, sum_ref, tot_ref, out_ref):
    h = jnp.zeros((_SQ, _SQ), jnp.float32)
    s = jnp.zeros((_SQ, _SQ), jnp.float32)
    for w in range(_NW):
        h = h + cnt_ref[w]
        s = s + sum_ref[w]
    rows = lax.broadcasted_iota(jnp.int32, (_SQ, _SQ), 0)
    cols = lax.broadcasted_iota(jnp.int32, (_SQ, _SQ), 1)
    binid = rows * _SQ + cols
    h = h - jnp.where(binid == 0, float(_PADN), 0.0)

    hi = jax.lax.Precision.HIGHEST
    inc = jnp.where(rows <= cols, 1.0, 0.0)
    strict = jnp.where(cols < rows, 1.0, 0.0)

    def csum(m):
        prefix = jax.lax.dot(m, inc, precision=hi)
        row_tot = jnp.sum(m, axis=1, keepdims=True)
        prev = jax.lax.dot(strict, row_tot, precision=hi)
        return prefix + prev

    csum_h = csum(h)
    csum_s = csum(s)
    cnt_ge = float(_REAL) - csum_h + h
    bstar = jnp.sum((cnt_ge >= float(_K)).astype(jnp.int32)) - 1
    sel = binid == bstar
    hb = jnp.sum(jnp.where(sel, h, 0.0))
    sb = jnp.sum(jnp.where(sel, s, 0.0))
    csum_hb = jnp.sum(jnp.where(sel, csum_h, 0.0))
    csum_sb = jnp.sum(jnp.where(sel, csum_s, 0.0))
    cnt_gt = float(_REAL) - csum_hb
    sum_gt = jnp.sum(s) - csum_sb
    r = float(_K) - cnt_gt
    vb = sb / jnp.maximum(hb, 1.0)
    topk_sum = sum_gt + r * vb
    out_ref[...] = tot_ref[...] / float(_REAL) + jnp.full((1, 1), topk_sum / float(_K))


def _select_call(cnt, sm, tot):
    return pl.pallas_call(
        _select_body,
        out_shape=jax.ShapeDtypeStruct((1, 1), jnp.float32),
    )(cnt, sm, tot)


def kernel(input, target):
    xt = input.T
    t2 = target.reshape(1, _N)
    loss, tot = _loss_call(xt, t2)
    cnt, sm = _make_hist_call()(loss)
    res = _select_call(cnt, sm, tot)
    return res[0, 0]
